# Initial kernel scaffold; baseline (speedup 1.0000x reference)
import jax
import jax.numpy as jnp
from jax import lax
from jax.experimental import pallas as pl
from jax.experimental.pallas import tpu as pltpu
from jax.experimental.pallas import tpu_sc as plsc

N_USER = 10000
N_ITEM = 10000
D = 128
E = 160000

L = 16
NC = 2
NS = 16
B = 128
ROWS = E // B
ROWS_PER_SC = ROWS // NC
RPT = ROWS_PER_SC // NS
N_OUT = 10000
OPT = N_OUT // NS
ZCHUNK = 125

RBLK = 1000


def _sc_body(xu, xi, s0, d0, s1, d1, s2, d2, sums, cnts,
             accum, cnt, src_idx, dst_idx, rows, ones_v, zrows, zcnt, sem):
    c = lax.axis_index("c")
    s = lax.axis_index("s")

    one16 = jnp.ones((L,), jnp.float32)
    zero16 = jnp.zeros((L,), jnp.float32)

    @pl.loop(0, B)
    def _(i):
        ones_v[i, :] = one16

    @pl.loop(0, ZCHUNK)
    def _(i):
        @pl.loop(0, D // L)
        def _(j):
            zrows[i, pl.ds(j * L, L)] = zero16

    @pl.loop(0, OPT)
    def _(i):
        zcnt[i, :] = zero16

    is_last = s == NS - 1
    nb = jnp.where(is_last, RPT + 1, RPT)
    base_row = c * ROWS_PER_SC + s * RPT
    tail_row = c * ROWS_PER_SC + (ROWS_PER_SC - 1)
    my_out = s * OPT

    for e, (src_hbm, dst_hbm, table) in enumerate(
            ((s0, d0, xu), (s1, d1, xi), (s2, d2, xu))):
        @pl.loop(0, OPT // ZCHUNK)
        def _(z):
            pltpu.sync_copy(zrows, accum.at[pl.ds(my_out + z * ZCHUNK, ZCHUNK)])
        pltpu.sync_copy(zcnt, cnt.at[pl.ds(my_out, OPT)])

        pltpu.sync_copy(src_hbm.at[pl.ds(base_row, RPT)], src_idx.at[pl.ds(0, RPT)])
        pltpu.sync_copy(dst_hbm.at[pl.ds(base_row, RPT)], dst_idx.at[pl.ds(0, RPT)])

        @pl.when(is_last)
        def _():
            pltpu.sync_copy(src_hbm.at[pl.ds(tail_row, 1)], src_idx.at[pl.ds(RPT, 1)])
            pltpu.sync_copy(dst_hbm.at[pl.ds(tail_row, 1)], dst_idx.at[pl.ds(RPT, 1)])

        plsc.subcore_barrier()

        @pl.loop(0, nb)
        def _(k):
            pltpu.async_copy(table.at[src_idx.at[k]], rows, sem).wait()
            pltpu.sync_copy(rows, accum.at[dst_idx.at[k]], add=True)
            pltpu.sync_copy(ones_v, cnt.at[dst_idx.at[k]], add=True)

        plsc.subcore_barrier()

        pltpu.sync_copy(accum.at[pl.ds(my_out, OPT)],
                        sums.at[e, c, pl.ds(my_out, OPT)])
        pltpu.sync_copy(cnt.at[pl.ds(my_out, OPT)],
                        cnts.at[e, c, pl.ds(my_out, OPT)])


def _combine_body(sums_ref, cnts_ref, wc_ref, bc_ref, wb_ref, bb_ref,
                  wf_ref, bf_ref, out_ref):
    sm = sums_ref[...]
    cl = cnts_ref[...]
    stot = sm[:, 0] + sm[:, 1]
    ct = cl[:, 0, :, 0:1] + cl[:, 1, :, 0:1]
    mean = stot / jnp.maximum(ct, 1.0)
    mask = (ct > 0).astype(jnp.float32)

    def lin(m, w_ref, b_ref, mk):
        y = lax.dot_general(m, w_ref[...], (((1,), (1,)), ((), ())),
                            precision=lax.Precision.HIGHEST)
        return y + b_ref[...] * mk

    out_ref[0] = (lin(mean[1], wb_ref, bb_ref, mask[1])
                  + lin(mean[2], wf_ref, bf_ref, mask[2]))
    out_ref[1] = lin(mean[0], wc_ref, bc_ref, mask[0])


def kernel(x_user, x_item, edge_click, edge_clicked_by, edge_follow,
           W_click, b_click, W_clicked_by, b_clicked_by, W_follow, b_follow):
    ec = edge_click.astype(jnp.int32)
    eb = edge_clicked_by.astype(jnp.int32)
    ef = edge_follow.astype(jnp.int32)
    s0, d0 = ec[0].reshape(ROWS, B), ec[1].reshape(ROWS, B)
    s1, d1 = eb[0].reshape(ROWS, B), eb[1].reshape(ROWS, B)
    s2, d2 = ef[0].reshape(ROWS, B), ef[1].reshape(ROWS, B)

    mesh = plsc.VectorSubcoreMesh(core_axis_name="c", subcore_axis_name="s",
                                  num_cores=NC, num_subcores=NS)
    sums, cnts = pl.kernel(
        _sc_body,
        out_type=(jax.ShapeDtypeStruct((3, NC, N_OUT, D), jnp.float32),
                  jax.ShapeDtypeStruct((3, NC, N_OUT, L), jnp.float32)),
        mesh=mesh,
        scratch_types=[
            pltpu.VMEM_SHARED((N_OUT, D), jnp.float32),
            pltpu.VMEM_SHARED((N_OUT, L), jnp.float32),
            pltpu.VMEM((RPT + 1, B), jnp.int32),
            pltpu.VMEM((RPT + 1, B), jnp.int32),
            pltpu.VMEM((B, D), jnp.float32),
            pltpu.VMEM((B, L), jnp.float32),
            pltpu.VMEM((ZCHUNK, D), jnp.float32),
            pltpu.VMEM((OPT, L), jnp.float32),
            pltpu.SemaphoreType.DMA,
        ],
    )(x_user, x_item, s0, d0, s1, d1, s2, d2)

    out2 = pl.pallas_call(
        _combine_body,
        grid=(N_OUT // RBLK,),
        in_specs=[
            pl.BlockSpec((3, NC, RBLK, D), lambda g: (0, 0, g, 0)),
            pl.BlockSpec((3, NC, RBLK, L), lambda g: (0, 0, g, 0)),
            pl.BlockSpec((D, D), lambda g: (0, 0)),
            pl.BlockSpec((1, D), lambda g: (0, 0)),
            pl.BlockSpec((D, D), lambda g: (0, 0)),
            pl.BlockSpec((1, D), lambda g: (0, 0)),
            pl.BlockSpec((D, D), lambda g: (0, 0)),
            pl.BlockSpec((1, D), lambda g: (0, 0)),
        ],
        out_specs=pl.BlockSpec((2, RBLK, D), lambda g: (0, g, 0)),
        out_shape=jax.ShapeDtypeStruct((2, N_OUT, D), jnp.float32),
    )(sums, cnts, W_click, b_click.reshape(1, D),
      W_clicked_by, b_clicked_by.reshape(1, D),
      W_follow, b_follow.reshape(1, D))

    return out2.reshape(2 * N_OUT, D)


# speedup vs baseline: 6.3889x; 6.3889x over previous
import jax
import jax.numpy as jnp
from jax import lax
from jax.experimental import pallas as pl
from jax.experimental.pallas import tpu as pltpu
from jax.experimental.pallas import tpu_sc as plsc

N_USER = 10000
N_ITEM = 10000
D = 128
E = 160000

L = 16
NC = 2
NS = 16
B = 128
ROWS = E // B
ROWS_PER_SC = ROWS // NC
RPT = ROWS_PER_SC // NS
N_OUT = 10000
OPT = N_OUT // NS
ZCHUNK = 25
ZCCHUNK = 125

RBLK = 1000


def _sc_body(xu, xi, s0, d0, s1, d1, s2, d2, sums, cnts,
             accum, cnt, src_idx, dst_idx, rows, ones_v, zrows, zcnt, sem):
    c = lax.axis_index("c")
    s = lax.axis_index("s")

    one16 = jnp.ones((L,), jnp.float32)
    zero16 = jnp.zeros((L,), jnp.float32)

    @pl.loop(0, B)
    def _(i):
        ones_v[i, :] = one16

    @pl.loop(0, ZCHUNK)
    def _(i):
        @pl.loop(0, D // L)
        def _(j):
            zrows[i, pl.ds(j * L, L)] = zero16

    @pl.loop(0, ZCCHUNK)
    def _(i):
        zcnt[i, :] = zero16

    is_last = s == NS - 1
    nb = jnp.where(is_last, RPT + 1, RPT)
    base_row = c * ROWS_PER_SC + s * RPT
    tail_row = c * ROWS_PER_SC + (ROWS_PER_SC - 1)
    my_out = s * OPT

    for e, (src_hbm, dst_hbm, table) in enumerate(
            ((s0, d0, xu), (s1, d1, xi), (s2, d2, xu))):
        @pl.loop(0, OPT // ZCHUNK)
        def _(z):
            pltpu.sync_copy(zrows, accum.at[pl.ds(my_out + z * ZCHUNK, ZCHUNK)])

        @pl.loop(0, OPT // ZCCHUNK)
        def _(z):
            pltpu.sync_copy(zcnt, cnt.at[pl.ds(my_out + z * ZCCHUNK, ZCCHUNK)])

        pltpu.sync_copy(src_hbm.at[pl.ds(base_row, RPT)], src_idx.at[pl.ds(0, RPT)])
        pltpu.sync_copy(dst_hbm.at[pl.ds(base_row, RPT)], dst_idx.at[pl.ds(0, RPT)])

        @pl.when(is_last)
        def _():
            pltpu.sync_copy(src_hbm.at[pl.ds(tail_row, 1)], src_idx.at[pl.ds(RPT, 1)])
            pltpu.sync_copy(dst_hbm.at[pl.ds(tail_row, 1)], dst_idx.at[pl.ds(RPT, 1)])

        plsc.subcore_barrier()

        @pl.loop(0, nb)
        def _(k):
            pltpu.async_copy(table.at[src_idx.at[k]], rows, sem).wait()
            pltpu.sync_copy(rows, accum.at[dst_idx.at[k]], add=True)
            pltpu.sync_copy(ones_v, cnt.at[dst_idx.at[k]], add=True)

        plsc.subcore_barrier()

        pltpu.sync_copy(accum.at[pl.ds(my_out, OPT)],
                        sums.at[e, c, pl.ds(my_out, OPT)])
        pltpu.sync_copy(cnt.at[pl.ds(my_out, OPT)],
                        cnts.at[e, c, pl.ds(my_out, OPT)])


def _combine_body(sums_ref, cnts_ref, wc_ref, bc_ref, wb_ref, bb_ref,
                  wf_ref, bf_ref, out_ref):
    sm = sums_ref[...]
    cl = cnts_ref[...]
    stot = sm[:, 0] + sm[:, 1]
    ct = cl[:, 0, :, 0:1] + cl[:, 1, :, 0:1]
    mean = stot / jnp.maximum(ct, 1.0)
    mask = (ct > 0).astype(jnp.float32)

    def lin(m, w_ref, b_ref, mk):
        y = lax.dot_general(m, w_ref[...], (((1,), (1,)), ((), ())),
                            precision=lax.Precision.HIGHEST)
        return y + b_ref[...] * mk

    out_ref[0] = (lin(mean[1], wb_ref, bb_ref, mask[1])
                  + lin(mean[2], wf_ref, bf_ref, mask[2]))
    out_ref[1] = lin(mean[0], wc_ref, bc_ref, mask[0])


def kernel(x_user, x_item, edge_click, edge_clicked_by, edge_follow,
           W_click, b_click, W_clicked_by, b_clicked_by, W_follow, b_follow):
    ec = edge_click.astype(jnp.int32)
    eb = edge_clicked_by.astype(jnp.int32)
    ef = edge_follow.astype(jnp.int32)
    s0, d0 = ec[0].reshape(ROWS, B), ec[1].reshape(ROWS, B)
    s1, d1 = eb[0].reshape(ROWS, B), eb[1].reshape(ROWS, B)
    s2, d2 = ef[0].reshape(ROWS, B), ef[1].reshape(ROWS, B)

    mesh = plsc.VectorSubcoreMesh(core_axis_name="c", subcore_axis_name="s",
                                  num_cores=NC, num_subcores=NS)
    sums, cnts = pl.kernel(
        _sc_body,
        out_type=(jax.ShapeDtypeStruct((3, NC, N_OUT, D), jnp.float32),
                  jax.ShapeDtypeStruct((3, NC, N_OUT, L), jnp.float32)),
        mesh=mesh,
        compiler_params=pltpu.CompilerParams(use_tc_tiling_on_sc=False),
        scratch_types=[
            pltpu.VMEM_SHARED((N_OUT, D), jnp.float32),
            pltpu.VMEM_SHARED((N_OUT, L), jnp.float32),
            pltpu.VMEM((RPT + 1, B), jnp.int32),
            pltpu.VMEM((RPT + 1, B), jnp.int32),
            pltpu.VMEM((B, D), jnp.float32),
            pltpu.VMEM((B, L), jnp.float32),
            pltpu.VMEM((ZCHUNK, D), jnp.float32),
            pltpu.VMEM((ZCCHUNK, L), jnp.float32),
            pltpu.SemaphoreType.DMA,
        ],
    )(x_user, x_item, s0, d0, s1, d1, s2, d2)

    out2 = pl.pallas_call(
        _combine_body,
        grid=(N_OUT // RBLK,),
        in_specs=[
            pl.BlockSpec((3, NC, RBLK, D), lambda g: (0, 0, g, 0)),
            pl.BlockSpec((3, NC, RBLK, L), lambda g: (0, 0, g, 0)),
            pl.BlockSpec((D, D), lambda g: (0, 0)),
            pl.BlockSpec((1, D), lambda g: (0, 0)),
            pl.BlockSpec((D, D), lambda g: (0, 0)),
            pl.BlockSpec((1, D), lambda g: (0, 0)),
            pl.BlockSpec((D, D), lambda g: (0, 0)),
            pl.BlockSpec((1, D), lambda g: (0, 0)),
        ],
        out_specs=pl.BlockSpec((2, RBLK, D), lambda g: (0, g, 0)),
        out_shape=jax.ShapeDtypeStruct((2, N_OUT, D), jnp.float32),
    )(sums, cnts, W_click, b_click.reshape(1, D),
      W_clicked_by, b_clicked_by.reshape(1, D),
      W_follow, b_follow.reshape(1, D))

    return out2.reshape(2 * N_OUT, D)


# speedup vs baseline: 8.4226x; 1.3183x over previous
import jax
import jax.numpy as jnp
from jax import lax
from jax.experimental import pallas as pl
from jax.experimental.pallas import tpu as pltpu
from jax.experimental.pallas import tpu_sc as plsc

N_USER = 10000
N_ITEM = 10000
D = 128
E = 160000

L = 16
NC = 2
NS = 16
B = 80
ROWS = E // B
ROWS_PER_SC = ROWS // NC
RPT = ROWS_PER_SC // NS
REM = ROWS_PER_SC - NS * RPT
N_OUT = 10000
OPT = N_OUT // NS
ZCHUNK = 25
ZCCHUNK = 125
NPAIR = RPT // 2

RBLK = 1000


def _sc_body(xu, xi, s0, d0, s1, d1, s2, d2, sums, cnts,
             accum, cnt, src_idx, dst_idx, rows0, rows1, ones_v, zrows, zcnt,
             gsem0, gsem1, ssem):
    c = lax.axis_index("c")
    s = lax.axis_index("s")

    one16 = jnp.ones((L,), jnp.float32)
    zero16 = jnp.zeros((L,), jnp.float32)

    @pl.loop(0, B)
    def _(i):
        ones_v[i, :] = one16

    @pl.loop(0, ZCHUNK)
    def _(i):
        @pl.loop(0, D // L)
        def _(j):
            zrows[i, pl.ds(j * L, L)] = zero16

    @pl.loop(0, ZCCHUNK)
    def _(i):
        zcnt[i, :] = zero16

    has_extra = s >= NS - REM
    nb = jnp.where(has_extra, RPT + 1, RPT)
    base_row = c * ROWS_PER_SC + s * RPT
    extra_row = c * ROWS_PER_SC + NS * RPT + (s - (NS - REM))
    my_out = s * OPT

    for e, (src_hbm, dst_hbm, table) in enumerate(
            ((s0, d0, xu), (s1, d1, xi), (s2, d2, xu))):
        @pl.loop(0, OPT // ZCHUNK)
        def _(z):
            pltpu.sync_copy(zrows, accum.at[pl.ds(my_out + z * ZCHUNK, ZCHUNK)])

        @pl.loop(0, OPT // ZCCHUNK)
        def _(z):
            pltpu.sync_copy(zcnt, cnt.at[pl.ds(my_out + z * ZCCHUNK, ZCCHUNK)])

        pltpu.sync_copy(src_hbm.at[pl.ds(base_row, RPT)], src_idx.at[pl.ds(0, RPT)])
        pltpu.sync_copy(dst_hbm.at[pl.ds(base_row, RPT)], dst_idx.at[pl.ds(0, RPT)])

        @pl.when(has_extra)
        def _():
            pltpu.sync_copy(src_hbm.at[pl.ds(extra_row, 1)], src_idx.at[pl.ds(RPT, 1)])
            pltpu.sync_copy(dst_hbm.at[pl.ds(extra_row, 1)], dst_idx.at[pl.ds(RPT, 1)])

        plsc.subcore_barrier()

        def gather(k, buf, sem):
            return pltpu.async_copy(table.at[src_idx.at[k]], buf, sem)

        def gather_wait(k, buf, sem):
            pltpu.make_async_copy(table.at[src_idx.at[k]], buf, sem).wait()

        def drain(k, buf):
            pltpu.async_copy(buf, accum.at[dst_idx.at[k]], ssem, add=True)
            pltpu.sync_copy(ones_v, cnt.at[dst_idx.at[k]], add=True)
            pltpu.make_async_copy(buf, accum.at[dst_idx.at[k]], ssem).wait()

        gather(0, rows0, gsem0)
        gather(1, rows1, gsem1)

        @pl.loop(0, NPAIR)
        def _(j):
            k0 = 2 * j
            k1 = k0 + 1
            gather_wait(k0, rows0, gsem0)
            drain(k0, rows0)

            @pl.when(k0 + 2 < nb)
            def _():
                gather(k0 + 2, rows0, gsem0)

            gather_wait(k1, rows1, gsem1)
            drain(k1, rows1)

            @pl.when(k1 + 2 < nb)
            def _():
                gather(k1 + 2, rows1, gsem1)

        @pl.when(has_extra)
        def _():
            gather_wait(RPT, rows0, gsem0)
            drain(RPT, rows0)

        plsc.subcore_barrier()

        pltpu.sync_copy(accum.at[pl.ds(my_out, OPT)],
                        sums.at[e, c, pl.ds(my_out, OPT)])
        pltpu.sync_copy(cnt.at[pl.ds(my_out, OPT)],
                        cnts.at[e, c, pl.ds(my_out, OPT)])


def _combine_body(sums_ref, cnts_ref, wc_ref, bc_ref, wb_ref, bb_ref,
                  wf_ref, bf_ref, out_ref):
    sm = sums_ref[...]
    cl = cnts_ref[...]
    stot = sm[:, 0] + sm[:, 1]
    ct = cl[:, 0, :, 0:1] + cl[:, 1, :, 0:1]
    mean = stot / jnp.maximum(ct, 1.0)
    mask = (ct > 0).astype(jnp.float32)

    def lin(m, w_ref, b_ref, mk):
        y = lax.dot_general(m, w_ref[...], (((1,), (1,)), ((), ())),
                            precision=lax.Precision.HIGHEST)
        return y + b_ref[...] * mk

    out_ref[0] = (lin(mean[1], wb_ref, bb_ref, mask[1])
                  + lin(mean[2], wf_ref, bf_ref, mask[2]))
    out_ref[1] = lin(mean[0], wc_ref, bc_ref, mask[0])


def kernel(x_user, x_item, edge_click, edge_clicked_by, edge_follow,
           W_click, b_click, W_clicked_by, b_clicked_by, W_follow, b_follow):
    ec = edge_click.astype(jnp.int32)
    eb = edge_clicked_by.astype(jnp.int32)
    ef = edge_follow.astype(jnp.int32)
    s0, d0 = ec[0].reshape(ROWS, B), ec[1].reshape(ROWS, B)
    s1, d1 = eb[0].reshape(ROWS, B), eb[1].reshape(ROWS, B)
    s2, d2 = ef[0].reshape(ROWS, B), ef[1].reshape(ROWS, B)

    mesh = plsc.VectorSubcoreMesh(core_axis_name="c", subcore_axis_name="s",
                                  num_cores=NC, num_subcores=NS)
    sums, cnts = pl.kernel(
        _sc_body,
        out_type=(jax.ShapeDtypeStruct((3, NC, N_OUT, D), jnp.float32),
                  jax.ShapeDtypeStruct((3, NC, N_OUT, L), jnp.float32)),
        mesh=mesh,
        compiler_params=pltpu.CompilerParams(use_tc_tiling_on_sc=False),
        scratch_types=[
            pltpu.VMEM_SHARED((N_OUT, D), jnp.float32),
            pltpu.VMEM_SHARED((N_OUT, L), jnp.float32),
            pltpu.VMEM((RPT + 1, B), jnp.int32),
            pltpu.VMEM((RPT + 1, B), jnp.int32),
            pltpu.VMEM((B, D), jnp.float32),
            pltpu.VMEM((B, D), jnp.float32),
            pltpu.VMEM((B, L), jnp.float32),
            pltpu.VMEM((ZCHUNK, D), jnp.float32),
            pltpu.VMEM((ZCCHUNK, L), jnp.float32),
            pltpu.SemaphoreType.DMA,
            pltpu.SemaphoreType.DMA,
            pltpu.SemaphoreType.DMA,
        ],
    )(x_user, x_item, s0, d0, s1, d1, s2, d2)

    out2 = pl.pallas_call(
        _combine_body,
        grid=(N_OUT // RBLK,),
        in_specs=[
            pl.BlockSpec((3, NC, RBLK, D), lambda g: (0, 0, g, 0)),
            pl.BlockSpec((3, NC, RBLK, L), lambda g: (0, 0, g, 0)),
            pl.BlockSpec((D, D), lambda g: (0, 0)),
            pl.BlockSpec((1, D), lambda g: (0, 0)),
            pl.BlockSpec((D, D), lambda g: (0, 0)),
            pl.BlockSpec((1, D), lambda g: (0, 0)),
            pl.BlockSpec((D, D), lambda g: (0, 0)),
            pl.BlockSpec((1, D), lambda g: (0, 0)),
        ],
        out_specs=pl.BlockSpec((2, RBLK, D), lambda g: (0, g, 0)),
        out_shape=jax.ShapeDtypeStruct((2, N_OUT, D), jnp.float32),
    )(sums, cnts, W_click, b_click.reshape(1, D),
      W_clicked_by, b_clicked_by.reshape(1, D),
      W_follow, b_follow.reshape(1, D))

    return out2.reshape(2 * N_OUT, D)
